# baseline (speedup 1.0000x reference)
import math

import jax
import jax.numpy as jnp
from jax.experimental import pallas as pl
from jax.experimental.pallas import tpu as pltpu

_W = 128
_BQ = 512
_THETA_LOG2 = 16.0
_TWO_PI = 2.0 * math.pi


def _rope_cos_sin(pos, hd):
    d = jax.lax.broadcasted_iota(jnp.int32, (1, hd), 1)
    q = ((d // 2) * 2).astype(jnp.float32)
    freqs = jnp.exp2(-(_THETA_LOG2 / hd) * q) / _TWO_PI
    phases = pos * freqs
    ph = (phases - jnp.floor(phases)) * _TWO_PI
    return jnp.cos(ph), jnp.sin(ph)


def _rot_matrix(hd):
    r = jax.lax.broadcasted_iota(jnp.int32, (hd, hd), 0)
    c = jax.lax.broadcasted_iota(jnp.int32, (hd, hd), 1)
    c_even = (c % 2) == 0
    m = jnp.where((r == c + 1) & c_even, -1.0, 0.0)
    m = jnp.where((r == c - 1) & ~c_even, 1.0, m)
    return m.astype(jnp.float32)


def _attn_kernel(q_ref, v_ref, o_ref,
                 cos_ref, sin_ref, mc_ref, mh_ref, kh_ref, vh_ref):
    i = pl.program_id(0)
    b = pl.program_id(1) * pl.num_programs(2) + pl.program_id(2)
    hd = q_ref.shape[-1]

    @pl.when((i == 0) & (b == 0))
    def _():
        iq = jax.lax.broadcasted_iota(jnp.int32, (_BQ, _BQ), 0)
        jc = jax.lax.broadcasted_iota(jnp.int32, (_BQ, _BQ), 1)
        mc_ref[...] = ((jc < iq) & (jc >= iq - _W)).astype(jnp.bfloat16)
        iqh = jax.lax.broadcasted_iota(jnp.int32, (_BQ, _W), 0)
        jh = jax.lax.broadcasted_iota(jnp.int32, (_BQ, _W), 1)
        mh_ref[...] = (jh >= iqh).astype(jnp.bfloat16)

    @pl.when(b == 0)
    def _():
        pos = i * _BQ + jax.lax.broadcasted_iota(jnp.int32, (_BQ, 1), 0)
        cos, sin = _rope_cos_sin(pos.astype(jnp.float32), hd)
        cos_ref[...] = cos
        sin_ref[...] = sin

    @pl.when(i == 0)
    def _():
        kh_ref[b] = jnp.zeros_like(kh_ref[b])
        vh_ref[b] = jnp.zeros_like(vh_ref[b])

    khist = kh_ref[b]
    vhist = vh_ref[b]

    qblk = q_ref[0, 0]
    qrot = jnp.dot(qblk, _rot_matrix(hd), preferred_element_type=jnp.float32)
    qr = qblk * cos_ref[...] + qrot * sin_ref[...]
    qr_bf = qr.astype(jnp.bfloat16)
    v_bf = v_ref[0, 0].astype(jnp.bfloat16)

    kh_ref[b] = qr_bf[_BQ - _W:]
    vh_ref[b] = v_bf[_BQ - _W:]

    s_cur = jax.lax.dot_general(
        qr_bf, qr_bf, (((1,), (1,)), ((), ())),
        preferred_element_type=jnp.float32,
    )
    s_hal = jax.lax.dot_general(
        qr_bf, khist, (((1,), (1,)), ((), ())),
        preferred_element_type=jnp.float32,
    )
    s_cur = s_cur.astype(jnp.bfloat16) * mc_ref[...]
    s_hal = s_hal.astype(jnp.bfloat16) * mh_ref[...]
    o_ref[0, 0] = (
        jax.lax.dot_general(
            s_cur, v_bf, (((1,), (0,)), ((), ())),
            preferred_element_type=jnp.float32,
        )
        + jax.lax.dot_general(
            s_hal, vhist, (((1,), (0,)), ((), ())),
            preferred_element_type=jnp.float32,
        )
    )


def kernel(Q, K, V):
    del K
    b, nh, t, hd = Q.shape
    bh = b * nh
    nblk = t // _BQ
    return pl.pallas_call(
        _attn_kernel,
        grid=(nblk, b, nh),
        in_specs=[
            pl.BlockSpec((1, 1, _BQ, hd), lambda i_, b_, h_: (b_, h_, i_, 0)),
            pl.BlockSpec((1, 1, _BQ, hd), lambda i_, b_, h_: (b_, h_, i_, 0)),
        ],
        out_specs=pl.BlockSpec((1, 1, _BQ, hd),
                               lambda i_, b_, h_: (b_, h_, i_, 0)),
        out_shape=jax.ShapeDtypeStruct((b, nh, t, hd), jnp.float32),
        scratch_shapes=[
            pltpu.VMEM((_BQ, hd), jnp.float32),
            pltpu.VMEM((_BQ, hd), jnp.float32),
            pltpu.VMEM((_BQ, _BQ), jnp.bfloat16),
            pltpu.VMEM((_BQ, _W), jnp.bfloat16),
            pltpu.VMEM((bh, _W, hd), jnp.bfloat16),
            pltpu.VMEM((bh, _W, hd), jnp.bfloat16),
        ],
        compiler_params=pltpu.CompilerParams(
            dimension_semantics=("arbitrary", "arbitrary", "arbitrary"),
        ),
    )(Q, V)


# speedup vs baseline: 1.2695x; 1.2695x over previous
import math

import jax
import jax.numpy as jnp
from jax.experimental import pallas as pl
from jax.experimental.pallas import tpu as pltpu

_W = 128
_BQ = 512
_THETA_LOG2 = 16.0
_TWO_PI = 2.0 * math.pi


def _rope_cos_sin(pos, hd):
    d = jax.lax.broadcasted_iota(jnp.int32, (1, hd), 1)
    q = ((d // 2) * 2).astype(jnp.float32)
    freqs = jnp.exp2(-(_THETA_LOG2 / hd) * q) / _TWO_PI
    phases = pos * freqs
    ph = (phases - jnp.floor(phases)) * _TWO_PI
    return jnp.cos(ph), jnp.sin(ph)


def _rot_matrix(hd):
    r = jax.lax.broadcasted_iota(jnp.int32, (hd, hd), 0)
    c = jax.lax.broadcasted_iota(jnp.int32, (hd, hd), 1)
    c_even = (c % 2) == 0
    m = jnp.where((r == c + 1) & c_even, -1.0, 0.0)
    m = jnp.where((r == c - 1) & ~c_even, 1.0, m)
    return m.astype(jnp.float32)


def _dot_nt(a, b):
    return jax.lax.dot_general(
        a, b, (((1,), (1,)), ((), ())), preferred_element_type=jnp.float32)


def _dot_nn(a, b):
    return jax.lax.dot_general(
        a, b, (((1,), (0,)), ((), ())), preferred_element_type=jnp.float32)


def _attn_kernel(q_ref, v_ref, o_ref,
                 cos_ref, sin_ref, mlow_ref, mgeq_ref, mband_ref,
                 kh_ref, vh_ref):
    i = pl.program_id(0)
    b = pl.program_id(1)
    hd = q_ref.shape[-1]

    @pl.when((i == 0) & (b == 0))
    def _():
        iq = jax.lax.broadcasted_iota(jnp.int32, (_W, _W), 0)
        jc = jax.lax.broadcasted_iota(jnp.int32, (_W, _W), 1)
        mlow_ref[...] = (jc < iq).astype(jnp.bfloat16)
        mgeq_ref[...] = (jc >= iq).astype(jnp.bfloat16)
        iqb = jax.lax.broadcasted_iota(jnp.int32, (_W, 2 * _W), 0)
        jb = jax.lax.broadcasted_iota(jnp.int32, (_W, 2 * _W), 1)
        mband_ref[...] = ((jb >= iqb) & (jb < iqb + _W)).astype(jnp.bfloat16)

    @pl.when(b == 0)
    def _():
        pos = i * _BQ + jax.lax.broadcasted_iota(jnp.int32, (_BQ, 1), 0)
        cos, sin = _rope_cos_sin(pos.astype(jnp.float32), hd)
        cos_ref[...] = cos
        sin_ref[...] = sin

    @pl.when(i == 0)
    def _():
        kh_ref[b] = jnp.zeros_like(kh_ref[b])
        vh_ref[b] = jnp.zeros_like(vh_ref[b])

    khist = kh_ref[b]
    vhist = vh_ref[b]

    qblk = q_ref[0]
    qrot = jnp.dot(qblk, _rot_matrix(hd), preferred_element_type=jnp.float32)
    qr = qblk * cos_ref[...] + qrot * sin_ref[...]
    qr_bf = qr.astype(jnp.bfloat16)
    v_bf = v_ref[0].astype(jnp.bfloat16)

    kh_ref[b] = qr_bf[_BQ - _W:]
    vh_ref[b] = v_bf[_BQ - _W:]

    q0 = qr_bf[0:_W]
    s_h = _dot_nt(q0, khist).astype(jnp.bfloat16) * mgeq_ref[...]
    s_c = _dot_nt(q0, q0).astype(jnp.bfloat16) * mlow_ref[...]
    o_ref[0, 0:_W] = _dot_nn(s_h, vhist) + _dot_nn(s_c, v_bf[0:_W])

    for j in range(1, _BQ // _W):
        qj = qr_bf[j * _W:(j + 1) * _W]
        keys = qr_bf[(j - 1) * _W:(j + 1) * _W]
        vj = v_bf[(j - 1) * _W:(j + 1) * _W]
        s = _dot_nt(qj, keys).astype(jnp.bfloat16) * mband_ref[...]
        o_ref[0, j * _W:(j + 1) * _W] = _dot_nn(s, vj)


def kernel(Q, K, V):
    del K
    b, nh, t, hd = Q.shape
    bh = b * nh
    q = Q.reshape(bh, t, hd)
    v = V.reshape(bh, t, hd)
    nblk = t // _BQ
    out = pl.pallas_call(
        _attn_kernel,
        grid=(nblk, bh),
        in_specs=[
            pl.BlockSpec((1, _BQ, hd), lambda i_, b_: (b_, i_, 0)),
            pl.BlockSpec((1, _BQ, hd), lambda i_, b_: (b_, i_, 0)),
        ],
        out_specs=pl.BlockSpec((1, _BQ, hd), lambda i_, b_: (b_, i_, 0)),
        out_shape=jax.ShapeDtypeStruct((bh, t, hd), jnp.float32),
        scratch_shapes=[
            pltpu.VMEM((_BQ, hd), jnp.float32),
            pltpu.VMEM((_BQ, hd), jnp.float32),
            pltpu.VMEM((_W, _W), jnp.bfloat16),
            pltpu.VMEM((_W, _W), jnp.bfloat16),
            pltpu.VMEM((_W, 2 * _W), jnp.bfloat16),
            pltpu.VMEM((bh, _W, hd), jnp.bfloat16),
            pltpu.VMEM((bh, _W, hd), jnp.bfloat16),
        ],
        compiler_params=pltpu.CompilerParams(
            dimension_semantics=("arbitrary", "arbitrary"),
        ),
    )(q, v)
    return out.reshape(b, nh, t, hd)


# speedup vs baseline: 1.2970x; 1.0216x over previous
import math

import jax
import jax.numpy as jnp
import numpy as np
from jax.experimental import pallas as pl
from jax.experimental.pallas import tpu as pltpu

_W = 128
_BQ = 512
_THETA_LOG2 = 16.0
_TWO_PI = 2.0 * math.pi


def _rope_tables(t, hd):
    d = np.arange(hd)
    q = (d // 2) * 2
    freqs = (2.0 ** (-(_THETA_LOG2 / hd) * q)) / _TWO_PI
    phases = np.arange(t)[:, None] * freqs[None, :]
    ph = (phases % 1.0) * _TWO_PI
    return (np.cos(ph).astype(np.float32), np.sin(ph).astype(np.float32))


def _rot_matrix(hd):
    r = jax.lax.broadcasted_iota(jnp.int32, (hd, hd), 0)
    c = jax.lax.broadcasted_iota(jnp.int32, (hd, hd), 1)
    c_even = (c % 2) == 0
    m = jnp.where((r == c + 1) & c_even, -1.0, 0.0)
    m = jnp.where((r == c - 1) & ~c_even, 1.0, m)
    return m.astype(jnp.float32)


def _dot_nt(a, b):
    return jax.lax.dot_general(
        a, b, (((1,), (1,)), ((), ())), preferred_element_type=jnp.float32)


def _dot_nn(a, b):
    return jax.lax.dot_general(
        a, b, (((1,), (0,)), ((), ())), preferred_element_type=jnp.float32)


def _attn_kernel(q_ref, v_ref, cos_ref, sin_ref, o_ref,
                 mlow_ref, mgeq_ref, mband_ref,
                 kh_ref, vh_ref):
    i = pl.program_id(0)
    b = pl.program_id(1)
    hd = q_ref.shape[-1]

    @pl.when((i == 0) & (b == 0))
    def _():
        iq = jax.lax.broadcasted_iota(jnp.int32, (_W, _W), 0)
        jc = jax.lax.broadcasted_iota(jnp.int32, (_W, _W), 1)
        mlow_ref[...] = (jc < iq).astype(jnp.bfloat16)
        mgeq_ref[...] = (jc >= iq).astype(jnp.bfloat16)
        iqb = jax.lax.broadcasted_iota(jnp.int32, (_W, 2 * _W), 0)
        jb = jax.lax.broadcasted_iota(jnp.int32, (_W, 2 * _W), 1)
        mband_ref[...] = ((jb >= iqb) & (jb < iqb + _W)).astype(jnp.bfloat16)

    @pl.when(i == 0)
    def _():
        kh_ref[b] = jnp.zeros_like(kh_ref[b])
        vh_ref[b] = jnp.zeros_like(vh_ref[b])

    khist = kh_ref[b]
    vhist = vh_ref[b]

    qblk = q_ref[0]
    qrot = jnp.dot(qblk, _rot_matrix(hd), preferred_element_type=jnp.float32)
    qr = qblk * cos_ref[...] + qrot * sin_ref[...]
    qr_bf = qr.astype(jnp.bfloat16)
    v_bf = v_ref[0].astype(jnp.bfloat16)

    kh_ref[b] = qr_bf[_BQ - _W:]
    vh_ref[b] = v_bf[_BQ - _W:]

    q0 = qr_bf[0:_W]
    s_h = _dot_nt(q0, khist).astype(jnp.bfloat16) * mgeq_ref[...]
    s_c = _dot_nt(q0, q0).astype(jnp.bfloat16) * mlow_ref[...]
    o_ref[0, 0:_W] = _dot_nn(s_h, vhist) + _dot_nn(s_c, v_bf[0:_W])

    for j in range(1, _BQ // _W):
        qj = qr_bf[j * _W:(j + 1) * _W]
        keys = qr_bf[(j - 1) * _W:(j + 1) * _W]
        vj = v_bf[(j - 1) * _W:(j + 1) * _W]
        s = _dot_nt(qj, keys).astype(jnp.bfloat16) * mband_ref[...]
        o_ref[0, j * _W:(j + 1) * _W] = _dot_nn(s, vj)


def kernel(Q, K, V):
    del K
    b, nh, t, hd = Q.shape
    bh = b * nh
    q = Q.reshape(bh, t, hd)
    v = V.reshape(bh, t, hd)
    nblk = t // _BQ
    cos_np, sin_np = _rope_tables(t, hd)
    cos_tab = jnp.asarray(cos_np)
    sin_tab = jnp.asarray(sin_np)
    out = pl.pallas_call(
        _attn_kernel,
        grid=(nblk, bh),
        in_specs=[
            pl.BlockSpec((1, _BQ, hd), lambda i_, b_: (b_, i_, 0)),
            pl.BlockSpec((1, _BQ, hd), lambda i_, b_: (b_, i_, 0)),
            pl.BlockSpec((_BQ, hd), lambda i_, b_: (i_, 0)),
            pl.BlockSpec((_BQ, hd), lambda i_, b_: (i_, 0)),
        ],
        out_specs=pl.BlockSpec((1, _BQ, hd), lambda i_, b_: (b_, i_, 0)),
        out_shape=jax.ShapeDtypeStruct((bh, t, hd), jnp.float32),
        scratch_shapes=[
            pltpu.VMEM((_W, _W), jnp.bfloat16),
            pltpu.VMEM((_W, _W), jnp.bfloat16),
            pltpu.VMEM((_W, 2 * _W), jnp.bfloat16),
            pltpu.VMEM((bh, _W, hd), jnp.bfloat16),
            pltpu.VMEM((bh, _W, hd), jnp.bfloat16),
        ],
        compiler_params=pltpu.CompilerParams(
            dimension_semantics=("arbitrary", "arbitrary"),
        ),
    )(q, v, cos_tab, sin_tab)
    return out.reshape(b, nh, t, hd)


# speedup vs baseline: 1.8619x; 1.4356x over previous
import math

import jax
import jax.numpy as jnp
import numpy as np
from jax.experimental import pallas as pl
from jax.experimental.pallas import tpu as pltpu

_W = 128
_THETA_LOG2 = 16.0
_TWO_PI = 2.0 * math.pi


def _rope_tables(t, hd):
    d = np.arange(hd)
    q = (d // 2) * 2
    freqs = (2.0 ** (-(_THETA_LOG2 / hd) * q)) / _TWO_PI
    phases = np.arange(t)[:, None] * freqs[None, :]
    ph = (phases % 1.0) * _TWO_PI
    return (np.cos(ph).astype(np.float32), np.sin(ph).astype(np.float32))


def _rot_matrix(hd):
    r = jax.lax.broadcasted_iota(jnp.int32, (hd, hd), 0)
    c = jax.lax.broadcasted_iota(jnp.int32, (hd, hd), 1)
    c_even = (c % 2) == 0
    m = jnp.where((r == c + 1) & c_even, -1.0, 0.0)
    m = jnp.where((r == c - 1) & ~c_even, 1.0, m)
    return m.astype(jnp.float32)


def _dot_nt(a, b):
    return jax.lax.dot_general(
        a, b, (((1,), (1,)), ((), ())), preferred_element_type=jnp.float32)


def _dot_nn(a, b):
    return jax.lax.dot_general(
        a, b, (((1,), (0,)), ((), ())), preferred_element_type=jnp.float32)


def _attn_kernel(q_ref, v_ref, cos_ref, sin_ref, o_ref, mband_ref):
    bb = pl.program_id(0)
    hh = pl.program_id(1)
    t = q_ref.shape[2]
    hd = q_ref.shape[-1]

    @pl.when((bb == 0) & (hh == 0))
    def _():
        iqb = jax.lax.broadcasted_iota(jnp.int32, (_W, 2 * _W), 0)
        jb = jax.lax.broadcasted_iota(jnp.int32, (_W, 2 * _W), 1)
        mband_ref[...] = ((jb >= iqb) & (jb < iqb + _W)).astype(jnp.bfloat16)

    qblk = q_ref[0, 0]
    qrot = jnp.dot(qblk, _rot_matrix(hd), preferred_element_type=jnp.float32)
    qr = qblk * cos_ref[...] + qrot * sin_ref[...]
    qr_bf = qr.astype(jnp.bfloat16)
    v_bf = v_ref[0, 0].astype(jnp.bfloat16)

    q0 = qr_bf[0:_W]
    s0 = _dot_nt(q0, q0).astype(jnp.bfloat16) * mband_ref[:, _W:]
    o_ref[0, 0, 0:_W] = _dot_nn(s0, v_bf[0:_W])

    for j in range(1, t // _W):
        qj = qr_bf[j * _W:(j + 1) * _W]
        keys = qr_bf[(j - 1) * _W:(j + 1) * _W]
        vj = v_bf[(j - 1) * _W:(j + 1) * _W]
        s = _dot_nt(qj, keys).astype(jnp.bfloat16) * mband_ref[...]
        o_ref[0, 0, j * _W:(j + 1) * _W] = _dot_nn(s, vj)


def kernel(Q, K, V):
    del K
    b, nh, t, hd = Q.shape
    cos_np, sin_np = _rope_tables(t, hd)
    cos_tab = jnp.asarray(cos_np)
    sin_tab = jnp.asarray(sin_np)
    return pl.pallas_call(
        _attn_kernel,
        grid=(b, nh),
        in_specs=[
            pl.BlockSpec((1, 1, t, hd), lambda b_, h_: (b_, h_, 0, 0)),
            pl.BlockSpec((1, 1, t, hd), lambda b_, h_: (b_, h_, 0, 0)),
            pl.BlockSpec((t, hd), lambda b_, h_: (0, 0)),
            pl.BlockSpec((t, hd), lambda b_, h_: (0, 0)),
        ],
        out_specs=pl.BlockSpec((1, 1, t, hd), lambda b_, h_: (b_, h_, 0, 0)),
        out_shape=jax.ShapeDtypeStruct((b, nh, t, hd), jnp.float32),
        scratch_shapes=[
            pltpu.VMEM((_W, 2 * _W), jnp.bfloat16),
        ],
        compiler_params=pltpu.CompilerParams(
            dimension_semantics=("arbitrary", "arbitrary"),
        ),
    )(Q, V, cos_tab, sin_tab)


# speedup vs baseline: 2.4282x; 1.3042x over previous
import math

import jax
import jax.numpy as jnp
import numpy as np
from jax.experimental import pallas as pl
from jax.experimental.pallas import tpu as pltpu

_W = 128
_THETA_LOG2 = 16.0
_TWO_PI = 2.0 * math.pi


def _rope_tables(t, hd):
    d = np.arange(hd)
    q = (d // 2) * 2
    freqs = (2.0 ** (-(_THETA_LOG2 / hd) * q)) / _TWO_PI
    phases = np.arange(t)[:, None] * freqs[None, :]
    ph = (phases % 1.0) * _TWO_PI
    return (np.cos(ph).astype(np.float32), np.sin(ph).astype(np.float32))


def _rot_matrix(hd):
    r = jax.lax.broadcasted_iota(jnp.int32, (hd, hd), 0)
    c = jax.lax.broadcasted_iota(jnp.int32, (hd, hd), 1)
    c_even = (c % 2) == 0
    m = jnp.where((r == c + 1) & c_even, -1.0, 0.0)
    m = jnp.where((r == c - 1) & ~c_even, 1.0, m)
    return m.astype(jnp.float32)


def _dot_nt(a, b):
    return jax.lax.dot_general(
        a, b, (((1,), (1,)), ((), ())), preferred_element_type=jnp.float32)


def _dot_nn(a, b):
    return jax.lax.dot_general(
        a, b, (((1,), (0,)), ((), ())), preferred_element_type=jnp.float32)


def _attn_kernel(q_ref, v_ref, cos_ref, sin_ref, o_ref, mband_ref):
    bb = pl.program_id(0)
    hh = pl.program_id(1)
    t = q_ref.shape[2]
    hd = q_ref.shape[-1]

    @pl.when((bb == 0) & (hh == 0))
    def _():
        iqb = jax.lax.broadcasted_iota(jnp.int32, (_W, 2 * _W), 0)
        jb = jax.lax.broadcasted_iota(jnp.int32, (_W, 2 * _W), 1)
        mband_ref[...] = ((jb >= iqb) & (jb < iqb + _W)).astype(jnp.bfloat16)

    qblk = q_ref[0]
    qrot = jnp.dot(qblk, _rot_matrix(hd), preferred_element_type=jnp.float32)
    qr = qblk * cos_ref[...] + qrot * sin_ref[...]
    qr_bf = qr.astype(jnp.bfloat16)
    v_bf = v_ref[0].astype(jnp.bfloat16)

    q0 = qr_bf[0:_W]
    s0 = _dot_nt(q0, q0).astype(jnp.bfloat16) * mband_ref[:, _W:]
    o_ref[0, 0, 0:_W] = _dot_nn(s0, v_bf[0:_W])

    for j in range(1, t // _W):
        qj = qr_bf[j * _W:(j + 1) * _W]
        keys = qr_bf[(j - 1) * _W:(j + 1) * _W]
        vj = v_bf[(j - 1) * _W:(j + 1) * _W]
        s = _dot_nt(qj, keys).astype(jnp.bfloat16) * mband_ref[...]
        o_ref[0, 0, j * _W:(j + 1) * _W] = _dot_nn(s, vj)


def kernel(Q, K, V):
    del K
    b, nh, t, hd = Q.shape
    cos_np, sin_np = _rope_tables(t, hd)
    cos_tab = jnp.asarray(cos_np)
    sin_tab = jnp.asarray(sin_np)
    bh = b * nh
    q = Q.reshape(bh, t, hd)
    v = V.reshape(bh, t, hd)
    return pl.pallas_call(
        _attn_kernel,
        grid=(b, nh),
        in_specs=[
            pl.BlockSpec((1, t, hd), lambda b_, h_: (b_ * nh + h_, 0, 0)),
            pl.BlockSpec((1, t, hd), lambda b_, h_: (b_ * nh + h_, 0, 0)),
            pl.BlockSpec((t, hd), lambda b_, h_: (0, 0)),
            pl.BlockSpec((t, hd), lambda b_, h_: (0, 0)),
        ],
        out_specs=pl.BlockSpec((1, 1, t, hd), lambda b_, h_: (b_, h_, 0, 0)),
        out_shape=jax.ShapeDtypeStruct((b, nh, t, hd), jnp.float32),
        scratch_shapes=[
            pltpu.VMEM((_W, 2 * _W), jnp.bfloat16),
        ],
        compiler_params=pltpu.CompilerParams(
            dimension_semantics=("arbitrary", "arbitrary"),
        ),
    )(q, v, cos_tab, sin_tab)


# speedup vs baseline: 2.4752x; 1.0193x over previous
import math

import jax
import jax.numpy as jnp
import numpy as np
from jax.experimental import pallas as pl
from jax.experimental.pallas import tpu as pltpu

_W = 128
_THETA_LOG2 = 16.0
_TWO_PI = 2.0 * math.pi


def _rope_tables(t, hd):
    d = np.arange(hd)
    q = (d // 2) * 2
    freqs = (2.0 ** (-(_THETA_LOG2 / hd) * q)) / _TWO_PI
    phases = np.arange(t)[:, None] * freqs[None, :]
    ph = (phases % 1.0) * _TWO_PI
    return (np.cos(ph).astype(np.float32), np.sin(ph).astype(np.float32))


def _rot_matrix(hd):
    r = jax.lax.broadcasted_iota(jnp.int32, (hd, hd), 0)
    c = jax.lax.broadcasted_iota(jnp.int32, (hd, hd), 1)
    c_even = (c % 2) == 0
    m = jnp.where((r == c + 1) & c_even, -1.0, 0.0)
    m = jnp.where((r == c - 1) & ~c_even, 1.0, m)
    return m.astype(jnp.float32)


def _dot_nt(a, b):
    return jax.lax.dot_general(
        a, b, (((1,), (1,)), ((), ())), preferred_element_type=jnp.float32)


def _dot_nn(a, b):
    return jax.lax.dot_general(
        a, b, (((1,), (0,)), ((), ())), preferred_element_type=jnp.float32)


def _attn_kernel(q_ref, v_ref, cos_ref, sin_ref, o_ref, mband_ref):
    t = q_ref.shape[1]
    hd = q_ref.shape[-1]

    @pl.when(pl.program_id(0) == 0)
    def _():
        iqb = jax.lax.broadcasted_iota(jnp.int32, (_W, 2 * _W), 0)
        jb = jax.lax.broadcasted_iota(jnp.int32, (_W, 2 * _W), 1)
        mband_ref[...] = ((jb >= iqb) & (jb < iqb + _W)).astype(jnp.bfloat16)

    qblk = q_ref[0]
    qrot = jnp.dot(qblk, _rot_matrix(hd), preferred_element_type=jnp.float32)
    qr = qblk * cos_ref[...] + qrot * sin_ref[...]
    qr_bf = qr.astype(jnp.bfloat16)
    v_bf = v_ref[0].astype(jnp.bfloat16)

    q0 = qr_bf[0:_W]
    s0 = _dot_nt(q0, q0).astype(jnp.bfloat16) * mband_ref[:, _W:]
    o_ref[0, 0:_W] = _dot_nn(s0, v_bf[0:_W])

    for j in range(1, t // _W):
        qj = qr_bf[j * _W:(j + 1) * _W]
        keys = qr_bf[(j - 1) * _W:(j + 1) * _W]
        vj = v_bf[(j - 1) * _W:(j + 1) * _W]
        s = _dot_nt(qj, keys).astype(jnp.bfloat16) * mband_ref[...]
        o_ref[0, j * _W:(j + 1) * _W] = _dot_nn(s, vj)


def kernel(Q, K, V):
    del K
    b, nh, t, hd = Q.shape
    cos_np, sin_np = _rope_tables(t, hd)
    cos_tab = jnp.asarray(cos_np)
    sin_tab = jnp.asarray(sin_np)
    bh = b * nh
    q = Q.reshape(bh, t, hd)
    v = V.reshape(bh, t, hd)
    out = pl.pallas_call(
        _attn_kernel,
        grid=(bh,),
        in_specs=[
            pl.BlockSpec((1, t, hd), lambda b_: (b_, 0, 0)),
            pl.BlockSpec((1, t, hd), lambda b_: (b_, 0, 0)),
            pl.BlockSpec((t, hd), lambda b_: (0, 0)),
            pl.BlockSpec((t, hd), lambda b_: (0, 0)),
        ],
        out_specs=pl.BlockSpec((1, t, hd), lambda b_: (b_, 0, 0)),
        out_shape=jax.ShapeDtypeStruct((bh, t, hd), jnp.float32),
        scratch_shapes=[
            pltpu.VMEM((_W, 2 * _W), jnp.bfloat16),
        ],
        compiler_params=pltpu.CompilerParams(
            dimension_semantics=("arbitrary",),
        ),
    )(q, v, cos_tab, sin_tab)
    return out.reshape(b, nh, t, hd)
